# baseline (speedup 1.0000x reference)
import math

import jax
import jax.numpy as jnp
from jax.experimental import pallas as pl
from jax.experimental.pallas import tpu as pltpu

_E = 8
_SEQ = 512
_PRED = 96
_FDIM = 128
_SCALES = (3, 7, 14)
_KL_LAMBDA = 0.001
_MAX_TIME = 200.0
_NFREQ = 4
_NB = 32
_EXT = _SEQ + 128


def _sh(d, j):
    r, width = d.shape
    if j == 0:
        return d
    if j > 0:
        z = jnp.zeros((r, j), dtype=d.dtype)
        return jnp.concatenate([d[:, j:], z], axis=1)
    z = jnp.zeros((r, -j), dtype=d.dtype)
    return jnp.concatenate([z, d[:, : width + j]], axis=1)


def _box_center(dx, w):
    if w == 3:
        return _sh(dx, -1) + dx + _sh(dx, 1)
    if w == 7:
        b3 = _sh(dx, -3) + _sh(dx, -2) + _sh(dx, -1)
        return b3 + dx + _sh(b3, 4)
    if w == 14:
        g3 = _sh(dx, -2) + _sh(dx, -1) + dx
        s7 = g3 + _sh(g3, -3) + _sh(dx, -6)
        return s7 + _sh(s7, 7)
    raise ValueError(w)


def _fold(long_ref, short_ref):
    ws = short_ref[...]
    acc = ws
    for c, w in enumerate(_SCALES):
        pad_l = w // 2
        pad_r = pad_l - (1 if w % 2 == 0 else 0)
        d_src = jnp.concatenate(
            [long_ref[e * len(_SCALES) + c] for e in range(_E)], axis=0
        )
        d = (d_src - ws) * (1.0 / w)
        dx = jnp.concatenate(
            [d, jnp.zeros((_E * _PRED, _EXT - _SEQ), d.dtype)], axis=1
        )
        vc = _box_center(dx, w)[:, :_SEQ]
        pref = d[:, 0:1]
        pcols = {0: pref}
        for t in range(1, pad_l):
            pref = pref + d[:, t : t + 1]
            pcols[t] = pref
        lcols = [jnp.zeros_like(d[:, 0:1])]
        for uu in range(1, pad_l + 1):
            lcols.append(pcols[pad_l - uu])
        lcorr = jnp.concatenate(
            lcols + [jnp.zeros((d.shape[0], _SEQ - pad_l - 1), d.dtype)], axis=1
        )
        suf = d[:, _SEQ - 1 : _SEQ]
        scols = {_SEQ - 1: suf}
        for t in range(_SEQ - 2, _SEQ - 1 - pad_r - pad_l, -1):
            suf = suf + d[:, t : t + 1]
            scols[t] = suf
        rcols = []
        for uu in range(_SEQ - 1 - pad_r, _SEQ - 1):
            m = 2 * (_SEQ - 1) - uu + pad_l
            rcols.append(scols[m - w + 1])
        rcols.append(jnp.zeros_like(d[:, 0:1]))
        rcorr = jnp.concatenate(
            [jnp.zeros((d.shape[0], _SEQ - 1 - pad_r), d.dtype)] + rcols, axis=1
        )
        acc = acc + (vc + lcorr + rcorr)
    return acc


def _fold_kernel(long_ref, short_ref, out_ref):
    out_ref[...] = _fold(long_ref, short_ref)


def _moe_kernel(
    x_ref, hh_ref, gw_ref, gb_ref, wf_ref, bt_ref,
    out_ref, kl_ref
):
    b = pl.program_id(0)
    pp = _NB * _PRED

    @pl.when(b == 0)
    def _():
        kl_ref[...] = jnp.zeros_like(kl_ref)

    hh = hh_ref[...].reshape(pp, 1)
    lanes_i = jax.lax.broadcasted_iota(jnp.int32, (pp, _E), 1)
    lanes_f = lanes_i.astype(jnp.float32)
    freq = jnp.where(lanes_f < _NFREQ, lanes_f + 1.0, lanes_f - (_NFREQ - 1.0))
    ang = (hh * (math.pi / _MAX_TIME)) * freq
    enc = jnp.where(lanes_f < _NFREQ, jnp.sin(ang), jnp.cos(ang))
    logits = jnp.concatenate(
        [
            jax.lax.dot_general(
                gw_ref[...],
                jnp.concatenate(
                    [
                        x_ref[n, _SEQ - _PRED :, :],
                        enc[n * _PRED : (n + 1) * _PRED],
                    ],
                    axis=1,
                ),
                (((1,), (1,)), ((), ())),
                preferred_element_type=jnp.float32,
            )
            for n in range(_NB)
        ],
        axis=1,
    )
    logits = logits + gb_ref[...]

    subl_i = jax.lax.broadcasted_iota(jnp.int32, (_E, pp), 0)
    m = jnp.max(logits, axis=0, keepdims=True)
    exl = jnp.exp(logits - m)
    se = jnp.sum(exl, axis=0, keepdims=True)
    g = exl / se
    lse = m + jnp.log(se)
    s_b = jnp.sum(logits) - _E * jnp.sum(lse)

    v1 = jnp.max(g, axis=0, keepdims=True)
    i1 = jnp.min(jnp.where(g == v1, subl_i, _E + 1), axis=0, keepdims=True)
    gm = jnp.where(subl_i == i1, -1.0, g)
    v2 = jnp.max(gm, axis=0, keepdims=True)
    i2 = jnp.min(jnp.where(gm == v2, subl_i, _E + 1), axis=0, keepdims=True)
    den = v1 + v2
    gst = (
        jnp.where(subl_i == i1, v1, 0.0) + jnp.where(subl_i == i2, v2, 0.0)
    ) / den
    gs = jnp.transpose(gst)
    bt_all = jnp.concatenate([bt_ref[...]] * _NB, axis=0)
    bias = jnp.sum(gs * bt_all, axis=1, keepdims=True)

    wfs = [wf_ref[e * _PRED : (e + 1) * _PRED] for e in range(_E)]
    for n in range(_NB):
        gsn = gs[n * _PRED : (n + 1) * _PRED]
        wfinal = gsn[:, 0:1] * wfs[0]
        for e in range(1, _E):
            wfinal = wfinal + gsn[:, e : e + 1] * wfs[e]
        out = jax.lax.dot_general(
            wfinal,
            x_ref[n],
            (((1,), (0,)), ((), ())),
            preferred_element_type=jnp.float32,
        )
        out_ref[n] = out + bias[n * _PRED : (n + 1) * _PRED]

    kl_ref[...] += jnp.full(kl_ref.shape, s_b, jnp.float32)


@jax.jit
def kernel(x, x_mark_enc, gate_W, gate_b, long_W, long_b, short_W, short_b):
    B = x.shape[0]
    short_r = short_W.reshape(_E * _PRED, _SEQ)
    long_r = long_W.reshape(_E * len(_SCALES), _PRED, _SEQ)
    hh3 = x_mark_enc[:, _SEQ - _PRED :, -1:]
    gb2 = gate_b.reshape(_E, 1)
    bt = (jnp.sum(long_b, axis=1) + short_b).T

    wfold = pl.pallas_call(
        _fold_kernel,
        out_shape=jax.ShapeDtypeStruct((_E * _PRED, _SEQ), jnp.float32),
    )(long_r, short_r)

    out, klacc = pl.pallas_call(
        _moe_kernel,
        grid=(B // _NB,),
        in_specs=[
            pl.BlockSpec((_NB, _SEQ, _FDIM), lambda b: (b, 0, 0)),
            pl.BlockSpec((_NB, _PRED, 1), lambda b: (b, 0, 0)),
            pl.BlockSpec((_E, _FDIM + 2 * _NFREQ), lambda b: (0, 0)),
            pl.BlockSpec((_E, 1), lambda b: (0, 0)),
            pl.BlockSpec((_E * _PRED, _SEQ), lambda b: (0, 0)),
            pl.BlockSpec((_PRED, _E), lambda b: (0, 0)),
        ],
        out_specs=[
            pl.BlockSpec((_NB, _PRED, _FDIM), lambda b: (b, 0, 0)),
            pl.BlockSpec((8, 128), lambda b: (0, 0)),
        ],
        out_shape=[
            jax.ShapeDtypeStruct((B, _PRED, _FDIM), jnp.float32),
            jax.ShapeDtypeStruct((8, 128), jnp.float32),
        ],
        compiler_params=pltpu.CompilerParams(
            dimension_semantics=("arbitrary",)
        ),
    )(x, hh3, gate_W, gb2, wfold, bt)

    kl = _KL_LAMBDA * _PRED * math.log(1.0 / _E) - (
        _KL_LAMBDA / (_E * B)
    ) * klacc[0, 0]
    return out, kl


# speedup vs baseline: 1.0628x; 1.0628x over previous
import math

import jax
import jax.numpy as jnp
from jax.experimental import pallas as pl
from jax.experimental.pallas import tpu as pltpu

_E = 8
_SEQ = 512
_PRED = 96
_FDIM = 128
_SCALES = (3, 7, 14)
_KL_LAMBDA = 0.001
_MAX_TIME = 200.0
_NFREQ = 4
_NB = 16
_EXT = _SEQ + 128


def _sh(d, j):
    r, width = d.shape
    if j == 0:
        return d
    if j > 0:
        z = jnp.zeros((r, j), dtype=d.dtype)
        return jnp.concatenate([d[:, j:], z], axis=1)
    z = jnp.zeros((r, -j), dtype=d.dtype)
    return jnp.concatenate([z, d[:, : width + j]], axis=1)


def _box_center(dx, w):
    if w == 3:
        return _sh(dx, -1) + dx + _sh(dx, 1)
    if w == 7:
        b3 = _sh(dx, -3) + _sh(dx, -2) + _sh(dx, -1)
        return b3 + dx + _sh(b3, 4)
    if w == 14:
        g3 = _sh(dx, -2) + _sh(dx, -1) + dx
        s7 = g3 + _sh(g3, -3) + _sh(dx, -6)
        return s7 + _sh(s7, 7)
    raise ValueError(w)


def _fold(long_ref, short_ref):
    ws = short_ref[...]
    acc = ws
    for c, w in enumerate(_SCALES):
        pad_l = w // 2
        pad_r = pad_l - (1 if w % 2 == 0 else 0)
        d_src = jnp.concatenate(
            [long_ref[e * len(_SCALES) + c] for e in range(_E)], axis=0
        )
        d = (d_src - ws) * (1.0 / w)
        dx = jnp.concatenate(
            [d, jnp.zeros((_E * _PRED, _EXT - _SEQ), d.dtype)], axis=1
        )
        vc = _box_center(dx, w)[:, :_SEQ]
        pref = d[:, 0:1]
        pcols = {0: pref}
        for t in range(1, pad_l):
            pref = pref + d[:, t : t + 1]
            pcols[t] = pref
        lcols = [jnp.zeros_like(d[:, 0:1])]
        for uu in range(1, pad_l + 1):
            lcols.append(pcols[pad_l - uu])
        lcorr = jnp.concatenate(
            lcols + [jnp.zeros((d.shape[0], _SEQ - pad_l - 1), d.dtype)], axis=1
        )
        suf = d[:, _SEQ - 1 : _SEQ]
        scols = {_SEQ - 1: suf}
        for t in range(_SEQ - 2, _SEQ - 1 - pad_r - pad_l, -1):
            suf = suf + d[:, t : t + 1]
            scols[t] = suf
        rcols = []
        for uu in range(_SEQ - 1 - pad_r, _SEQ - 1):
            m = 2 * (_SEQ - 1) - uu + pad_l
            rcols.append(scols[m - w + 1])
        rcols.append(jnp.zeros_like(d[:, 0:1]))
        rcorr = jnp.concatenate(
            [jnp.zeros((d.shape[0], _SEQ - 1 - pad_r), d.dtype)] + rcols, axis=1
        )
        acc = acc + (vc + lcorr + rcorr)
    return acc


def _moe_kernel(
    x_ref, hh_ref, gw_ref, gb_ref, long_ref, short_ref, bt_ref,
    out_ref, kl_ref, wf_ref
):
    b = pl.program_id(0)
    pp = _NB * _PRED

    @pl.when(b == 0)
    def _():
        wf_ref[...] = _fold(long_ref, short_ref)
        kl_ref[...] = jnp.zeros_like(kl_ref)

    hh = hh_ref[...].reshape(pp, 1)
    lanes_i = jax.lax.broadcasted_iota(jnp.int32, (pp, _E), 1)
    lanes_f = lanes_i.astype(jnp.float32)
    freq = jnp.where(lanes_f < _NFREQ, lanes_f + 1.0, lanes_f - (_NFREQ - 1.0))
    ang = (hh * (math.pi / _MAX_TIME)) * freq
    enc = jnp.where(lanes_f < _NFREQ, jnp.sin(ang), jnp.cos(ang))
    logits = jnp.concatenate(
        [
            jax.lax.dot_general(
                gw_ref[...],
                jnp.concatenate(
                    [
                        x_ref[n, _SEQ - _PRED :, :],
                        enc[n * _PRED : (n + 1) * _PRED],
                    ],
                    axis=1,
                ),
                (((1,), (1,)), ((), ())),
                preferred_element_type=jnp.float32,
            )
            for n in range(_NB)
        ],
        axis=1,
    )
    logits = logits + gb_ref[...]

    subl_i = jax.lax.broadcasted_iota(jnp.int32, (_E, pp), 0)
    m = jnp.max(logits, axis=0, keepdims=True)
    exl = jnp.exp(logits - m)
    se = jnp.sum(exl, axis=0, keepdims=True)
    g = exl / se
    lse = m + jnp.log(se)
    s_b = jnp.sum(logits) - _E * jnp.sum(lse)

    v1 = jnp.max(g, axis=0, keepdims=True)
    i1 = jnp.min(jnp.where(g == v1, subl_i, _E + 1), axis=0, keepdims=True)
    gm = jnp.where(subl_i == i1, -1.0, g)
    v2 = jnp.max(gm, axis=0, keepdims=True)
    i2 = jnp.min(jnp.where(gm == v2, subl_i, _E + 1), axis=0, keepdims=True)
    den = v1 + v2
    gst = (
        jnp.where(subl_i == i1, v1, 0.0) + jnp.where(subl_i == i2, v2, 0.0)
    ) / den
    gs = jnp.transpose(gst)
    bt_all = jnp.concatenate([bt_ref[...]] * _NB, axis=0)
    bias = jnp.sum(gs * bt_all, axis=1, keepdims=True)

    wfs = [wf_ref[e * _PRED : (e + 1) * _PRED] for e in range(_E)]
    for n in range(_NB):
        gsn = gs[n * _PRED : (n + 1) * _PRED]
        wfinal = gsn[:, 0:1] * wfs[0]
        for e in range(1, _E):
            wfinal = wfinal + gsn[:, e : e + 1] * wfs[e]
        out = jax.lax.dot_general(
            wfinal,
            x_ref[n],
            (((1,), (0,)), ((), ())),
            preferred_element_type=jnp.float32,
        )
        out_ref[n] = out + bias[n * _PRED : (n + 1) * _PRED]

    kl_ref[...] += jnp.full(kl_ref.shape, s_b, jnp.float32)


@jax.jit
def kernel(x, x_mark_enc, gate_W, gate_b, long_W, long_b, short_W, short_b):
    B = x.shape[0]
    short_r = short_W.reshape(_E * _PRED, _SEQ)
    long_r = long_W.reshape(_E * len(_SCALES), _PRED, _SEQ)
    hh3 = x_mark_enc[:, _SEQ - _PRED :, -1:]
    gb2 = gate_b.reshape(_E, 1)
    bt = (jnp.sum(long_b, axis=1) + short_b).T

    out, klacc = pl.pallas_call(
        _moe_kernel,
        grid=(B // _NB,),
        in_specs=[
            pl.BlockSpec((_NB, _SEQ, _FDIM), lambda b: (b, 0, 0)),
            pl.BlockSpec((_NB, _PRED, 1), lambda b: (b, 0, 0)),
            pl.BlockSpec((_E, _FDIM + 2 * _NFREQ), lambda b: (0, 0)),
            pl.BlockSpec((_E, 1), lambda b: (0, 0)),
            pl.BlockSpec(
                (_E * len(_SCALES), _PRED, _SEQ), lambda b: (0, 0, 0)
            ),
            pl.BlockSpec((_E * _PRED, _SEQ), lambda b: (0, 0)),
            pl.BlockSpec((_PRED, _E), lambda b: (0, 0)),
        ],
        out_specs=[
            pl.BlockSpec((_NB, _PRED, _FDIM), lambda b: (b, 0, 0)),
            pl.BlockSpec((8, 128), lambda b: (0, 0)),
        ],
        out_shape=[
            jax.ShapeDtypeStruct((B, _PRED, _FDIM), jnp.float32),
            jax.ShapeDtypeStruct((8, 128), jnp.float32),
        ],
        scratch_shapes=[pltpu.VMEM((_E * _PRED, _SEQ), jnp.float32)],
        compiler_params=pltpu.CompilerParams(
            dimension_semantics=("arbitrary",)
        ),
    )(x, hh3, gate_W, gb2, long_r, short_r, bt)

    kl = _KL_LAMBDA * _PRED * math.log(1.0 / _E) - (
        _KL_LAMBDA / (_E * B)
    ) * klacc[0, 0]
    return out, kl
